# baseline (speedup 1.0000x reference)
import functools

import numpy as np
import jax
import jax.numpy as jnp
from jax import lax
from jax.experimental import pallas as pl
from jax.experimental.pallas import tpu as pltpu
from jax.experimental.pallas import tpu_sc as plsc


def _norm_pdf_np(z):
    return np.exp(-0.5 * z * z) / np.sqrt(2.0 * np.pi)


def _select_indices(frame_count: int) -> list:
    rng = np.random.RandomState(0)
    num_segments = 1
    idxs = np.linspace(0, frame_count - 1, frame_count, dtype=int)
    if frame_count <= num_segments * 2:
        idxs = np.repeat(idxs, int(frame_count * num_segments / len(idxs)))
        frame_count *= num_segments
    seg_sizes = _norm_pdf_np(np.linspace(-1, 1, num_segments))
    seg_sizes = 1 - seg_sizes if frame_count > num_segments else seg_sizes
    seg_sizes = seg_sizes / seg_sizes.sum() * frame_count
    seg_sizes = seg_sizes.astype(int)
    choices = []
    last_idx = 0
    for i, seg_size in enumerate(seg_sizes):
        next_idx = last_idx + seg_size if i < len(seg_sizes) - 1 else None
        choices.append(int(rng.choice(idxs[last_idx:next_idx], 1)[0]))
        last_idx = next_idx
    return choices


@functools.lru_cache(maxsize=None)
def _make_sc_copy(frame_elems: int, choice: int, total_elems: int):
    info = plsc.get_sparse_core_info()
    nw = info.num_cores * info.num_subcores
    chunk = frame_elems // nw
    assert chunk * nw == frame_elems and chunk % 8 == 0
    mesh = plsc.VectorSubcoreMesh(core_axis_name="c", subcore_axis_name="s")

    @functools.partial(
        pl.kernel,
        mesh=mesh,
        out_type=jax.ShapeDtypeStruct((frame_elems,), jnp.float32),
        scratch_types=[pltpu.VMEM((chunk,), jnp.float32)],
    )
    def sc_copy(x_hbm, out_hbm, buf):
        wid = lax.axis_index("s") * info.num_cores + lax.axis_index("c")
        base = wid * chunk
        pltpu.sync_copy(x_hbm.at[pl.ds(choice * frame_elems + base, chunk)], buf)
        pltpu.sync_copy(buf, out_hbm.at[pl.ds(base, chunk)])

    return sc_copy


def kernel(x):
    frame_shape = x.shape[1:]
    frame_elems = int(np.prod(frame_shape))
    choices = _select_indices(x.shape[0])
    sc_copy = _make_sc_copy(frame_elems, choices[0], x.size)
    out = sc_copy(x.reshape(-1))
    return out.reshape((len(choices),) + frame_shape)


# speedup vs baseline: 1.8178x; 1.8178x over previous
import functools

import numpy as np
import jax
import jax.numpy as jnp
from jax import lax
from jax.experimental import pallas as pl
from jax.experimental.pallas import tpu as pltpu
from jax.experimental.pallas import tpu_sc as plsc


def _norm_pdf_np(z):
    return np.exp(-0.5 * z * z) / np.sqrt(2.0 * np.pi)


def _select_indices(frame_count: int) -> list:
    rng = np.random.RandomState(0)
    num_segments = 1
    idxs = np.linspace(0, frame_count - 1, frame_count, dtype=int)
    if frame_count <= num_segments * 2:
        idxs = np.repeat(idxs, int(frame_count * num_segments / len(idxs)))
        frame_count *= num_segments
    seg_sizes = _norm_pdf_np(np.linspace(-1, 1, num_segments))
    seg_sizes = 1 - seg_sizes if frame_count > num_segments else seg_sizes
    seg_sizes = seg_sizes / seg_sizes.sum() * frame_count
    seg_sizes = seg_sizes.astype(int)
    choices = []
    last_idx = 0
    for i, seg_size in enumerate(seg_sizes):
        next_idx = last_idx + seg_size if i < len(seg_sizes) - 1 else None
        choices.append(int(rng.choice(idxs[last_idx:next_idx], 1)[0]))
        last_idx = next_idx
    return choices


@functools.lru_cache(maxsize=None)
def _make_sc_copy(frame_shape: tuple, choice: int):
    info = plsc.get_sparse_core_info()
    mesh = plsc.VectorSubcoreMesh(core_axis_name="c", subcore_axis_name="s")

    @functools.partial(
        pl.kernel,
        mesh=mesh,
        out_type=jax.ShapeDtypeStruct((1,) + frame_shape, jnp.float32),
    )
    def sc_copy(x_hbm, out_hbm):
        wid = lax.axis_index("s") * info.num_cores + lax.axis_index("c")

        @pl.when(wid == 0)
        def _():
            pltpu.sync_copy(x_hbm.at[pl.ds(choice, 1)], out_hbm)

    return sc_copy


def kernel(x):
    frame_shape = x.shape[1:]
    choices = _select_indices(x.shape[0])
    sc_copy = _make_sc_copy(frame_shape, choices[0])
    return sc_copy(x)


# speedup vs baseline: 1.8249x; 1.0039x over previous
import functools

import numpy as np
import jax
import jax.numpy as jnp
from jax import lax
from jax.experimental import pallas as pl
from jax.experimental.pallas import tpu as pltpu
from jax.experimental.pallas import tpu_sc as plsc


def _norm_pdf_np(z):
    return np.exp(-0.5 * z * z) / np.sqrt(2.0 * np.pi)


def _select_indices(frame_count: int) -> list:
    rng = np.random.RandomState(0)
    num_segments = 1
    idxs = np.linspace(0, frame_count - 1, frame_count, dtype=int)
    if frame_count <= num_segments * 2:
        idxs = np.repeat(idxs, int(frame_count * num_segments / len(idxs)))
        frame_count *= num_segments
    seg_sizes = _norm_pdf_np(np.linspace(-1, 1, num_segments))
    seg_sizes = 1 - seg_sizes if frame_count > num_segments else seg_sizes
    seg_sizes = seg_sizes / seg_sizes.sum() * frame_count
    seg_sizes = seg_sizes.astype(int)
    choices = []
    last_idx = 0
    for i, seg_size in enumerate(seg_sizes):
        next_idx = last_idx + seg_size if i < len(seg_sizes) - 1 else None
        choices.append(int(rng.choice(idxs[last_idx:next_idx], 1)[0]))
        last_idx = next_idx
    return choices


@functools.lru_cache(maxsize=None)
def _make_sc_copy(frame_shape: tuple, choice: int):
    info = plsc.get_sparse_core_info()
    mesh = plsc.VectorSubcoreMesh(core_axis_name="c", subcore_axis_name="s")

    @functools.partial(
        pl.kernel,
        mesh=mesh,
        out_type=jax.ShapeDtypeStruct((1,) + frame_shape, jnp.float32),
        compiler_params=pltpu.CompilerParams(use_tc_tiling_on_sc=True),
    )
    def sc_copy(x_hbm, out_hbm):
        wid = lax.axis_index("s") * info.num_cores + lax.axis_index("c")

        @pl.when(wid == 0)
        def _():
            pltpu.sync_copy(x_hbm.at[pl.ds(choice, 1)], out_hbm)

    return sc_copy


def kernel(x):
    frame_shape = x.shape[1:]
    choices = _select_indices(x.shape[0])
    sc_copy = _make_sc_copy(frame_shape, choices[0])
    return sc_copy(x)


# speedup vs baseline: 2.2172x; 1.2150x over previous
import functools

import numpy as np
import jax
import jax.numpy as jnp
from jax import lax
from jax.experimental import pallas as pl
from jax.experimental.pallas import tpu as pltpu
from jax.experimental.pallas import tpu_sc as plsc


def _norm_pdf_np(z):
    return np.exp(-0.5 * z * z) / np.sqrt(2.0 * np.pi)


def _select_indices(frame_count: int) -> list:
    rng = np.random.RandomState(0)
    num_segments = 1
    idxs = np.linspace(0, frame_count - 1, frame_count, dtype=int)
    if frame_count <= num_segments * 2:
        idxs = np.repeat(idxs, int(frame_count * num_segments / len(idxs)))
        frame_count *= num_segments
    seg_sizes = _norm_pdf_np(np.linspace(-1, 1, num_segments))
    seg_sizes = 1 - seg_sizes if frame_count > num_segments else seg_sizes
    seg_sizes = seg_sizes / seg_sizes.sum() * frame_count
    seg_sizes = seg_sizes.astype(int)
    choices = []
    last_idx = 0
    for i, seg_size in enumerate(seg_sizes):
        next_idx = last_idx + seg_size if i < len(seg_sizes) - 1 else None
        choices.append(int(rng.choice(idxs[last_idx:next_idx], 1)[0]))
        last_idx = next_idx
    return choices


@functools.lru_cache(maxsize=None)
def _make_sc_copy(frame_shape: tuple, choice: int):
    info = plsc.get_sparse_core_info()
    mesh = plsc.VectorSubcoreMesh(core_axis_name="c", subcore_axis_name="s")

    @functools.partial(
        pl.kernel,
        mesh=mesh,
        out_type=jax.ShapeDtypeStruct((1,) + frame_shape, jnp.float32),
        compiler_params=pltpu.CompilerParams(use_tc_tiling_on_sc=True),
    )
    def sc_copy(x_hbm, out_hbm):
        wid = lax.axis_index("s") * info.num_cores + lax.axis_index("c")

        @pl.when(wid == 0)
        def _():
            pltpu.sync_copy(x_hbm.at[pl.ds(choice, 1)], out_hbm)

    return sc_copy


@functools.lru_cache(maxsize=None)
def _make_tc_copy(frame_shape: tuple, choice: int):
    blk = (1,) + frame_shape

    def body(x_ref, o_ref):
        o_ref[...] = x_ref[...]

    return pl.pallas_call(
        body,
        out_shape=jax.ShapeDtypeStruct(blk, jnp.float32),
        grid=(1,),
        in_specs=[pl.BlockSpec(blk, lambda i: (choice, 0, 0, 0))],
        out_specs=pl.BlockSpec(blk, lambda i: (0, 0, 0, 0)),
    )


def kernel(x):
    frame_shape = x.shape[1:]
    choices = _select_indices(x.shape[0])
    tc_copy = _make_tc_copy(frame_shape, choices[0])
    return tc_copy(x)


# speedup vs baseline: 7.5251x; 3.3940x over previous
import functools

import numpy as np
import jax
import jax.numpy as jnp
from jax import lax
from jax.experimental import pallas as pl
from jax.experimental.pallas import tpu as pltpu
from jax.experimental.pallas import tpu_sc as plsc


def _norm_pdf_np(z):
    return np.exp(-0.5 * z * z) / np.sqrt(2.0 * np.pi)


def _select_indices(frame_count: int) -> list:
    rng = np.random.RandomState(0)
    num_segments = 1
    idxs = np.linspace(0, frame_count - 1, frame_count, dtype=int)
    if frame_count <= num_segments * 2:
        idxs = np.repeat(idxs, int(frame_count * num_segments / len(idxs)))
        frame_count *= num_segments
    seg_sizes = _norm_pdf_np(np.linspace(-1, 1, num_segments))
    seg_sizes = 1 - seg_sizes if frame_count > num_segments else seg_sizes
    seg_sizes = seg_sizes / seg_sizes.sum() * frame_count
    seg_sizes = seg_sizes.astype(int)
    choices = []
    last_idx = 0
    for i, seg_size in enumerate(seg_sizes):
        next_idx = last_idx + seg_size if i < len(seg_sizes) - 1 else None
        choices.append(int(rng.choice(idxs[last_idx:next_idx], 1)[0]))
        last_idx = next_idx
    return choices


@functools.lru_cache(maxsize=None)
def _make_sc_gather(n_groups: int, ftiles: int, wsub: int, flanes: int,
                    ift: int, ilane: int):
    info = plsc.get_sparse_core_info()
    nw = info.num_cores * info.num_subcores
    while nw > 1 and (n_groups % nw != 0 or (n_groups // nw) % 8 != 0):
        nw -= 1
    chunk = n_groups // nw
    mesh = plsc.VectorSubcoreMesh(core_axis_name="c", subcore_axis_name="s")

    il8 = (ilane // 8) * 8
    lane_in_block = ilane - il8
    assert chunk % 16 == 0

    @functools.partial(
        pl.kernel,
        mesh=mesh,
        out_type=jax.ShapeDtypeStruct((wsub, n_groups), jnp.float32),
        scratch_types=[
            pltpu.VMEM((wsub, chunk, 8), jnp.float32),
            pltpu.VMEM((wsub, chunk), jnp.float32),
        ],
        compiler_params=pltpu.CompilerParams(
            use_tc_tiling_on_sc=False, needs_layout_passes=False
        ),
    )
    def sc_gather(x_hbm, out_hbm, blocks, col):
        wid = lax.axis_index("s") * info.num_cores + lax.axis_index("c")
        g0 = wid * chunk

        @pl.when(wid < nw)
        def _():
            for wi in range(wsub):
                pltpu.sync_copy(
                    x_hbm.at[pl.ds(g0, chunk), ift, wi, pl.ds(il8, 8)],
                    blocks.at[wi],
                )
            lane16 = lax.iota(jnp.int32, 16)
            col16 = jnp.full((16,), lane_in_block, jnp.int32)
            for wi in range(wsub):
                def body(j, _, wi=wi):
                    vals = plsc.load_gather(
                        blocks.at[wi], [j * 16 + lane16, col16]
                    )
                    col[wi, pl.ds(j * 16, 16)] = vals
                    return _
                lax.fori_loop(0, chunk // 16, body, 0)
            pltpu.sync_copy(col, out_hbm.at[:, pl.ds(g0, chunk)])

    return sc_gather


def kernel(x):
    frames, chn, hgt, wdt = x.shape
    choices = _select_indices(frames)
    ch = choices[0]
    flanes = 128
    wsub = 8
    ftiles = frames // flanes
    wtiles = wdt // wsub
    n_groups = chn * hgt * wtiles
    xv = (
        x.reshape(ftiles, flanes, chn, hgt, wtiles, wsub)
        .transpose(2, 3, 4, 0, 5, 1)
        .reshape(n_groups, ftiles, wsub, flanes)
    )
    sc_gather = _make_sc_gather(
        n_groups, ftiles, wsub, flanes, ch // flanes, ch % flanes
    )
    out2 = sc_gather(xv)
    out = out2.T.reshape(chn, hgt, wtiles, wsub).reshape(1, chn, hgt, wdt)
    return out


# speedup vs baseline: 11.1711x; 1.4845x over previous
import functools

import numpy as np
import jax
import jax.numpy as jnp
from jax import lax
from jax.experimental import pallas as pl
from jax.experimental.pallas import tpu as pltpu
from jax.experimental.pallas import tpu_sc as plsc


def _norm_pdf_np(z):
    return np.exp(-0.5 * z * z) / np.sqrt(2.0 * np.pi)


def _select_indices(frame_count: int) -> list:
    rng = np.random.RandomState(0)
    num_segments = 1
    idxs = np.linspace(0, frame_count - 1, frame_count, dtype=int)
    if frame_count <= num_segments * 2:
        idxs = np.repeat(idxs, int(frame_count * num_segments / len(idxs)))
        frame_count *= num_segments
    seg_sizes = _norm_pdf_np(np.linspace(-1, 1, num_segments))
    seg_sizes = 1 - seg_sizes if frame_count > num_segments else seg_sizes
    seg_sizes = seg_sizes / seg_sizes.sum() * frame_count
    seg_sizes = seg_sizes.astype(int)
    choices = []
    last_idx = 0
    for i, seg_size in enumerate(seg_sizes):
        next_idx = last_idx + seg_size if i < len(seg_sizes) - 1 else None
        choices.append(int(rng.choice(idxs[last_idx:next_idx], 1)[0]))
        last_idx = next_idx
    return choices


@functools.lru_cache(maxsize=None)
def _make_sc_gather(n_groups: int, ftiles: int, wsub: int, flanes: int,
                    ift: int, ilane: int):
    info = plsc.get_sparse_core_info()
    nw = info.num_cores * info.num_subcores
    while nw > 1 and (n_groups % nw != 0 or (n_groups // nw) % 8 != 0):
        nw -= 1
    chunk = n_groups // nw
    mesh = plsc.VectorSubcoreMesh(core_axis_name="c", subcore_axis_name="s")

    il8 = (ilane // 8) * 8
    lane_in_block = ilane - il8
    assert chunk % 16 == 0

    @functools.partial(
        pl.kernel,
        mesh=mesh,
        out_type=jax.ShapeDtypeStruct((n_groups, wsub), jnp.float32),
        scratch_types=[
            pltpu.VMEM((wsub, chunk, 8), jnp.float32),
            pltpu.VMEM((chunk, wsub), jnp.float32),
            pltpu.SemaphoreType.DMA,
        ],
        compiler_params=pltpu.CompilerParams(
            use_tc_tiling_on_sc=False, needs_layout_passes=False
        ),
    )
    def sc_gather(x_hbm, out_hbm, blocks, colt, sem):
        wid = lax.axis_index("s") * info.num_cores + lax.axis_index("c")
        g0 = wid * chunk

        @pl.when(wid < nw)
        def _():
            descs = [
                pltpu.make_async_copy(
                    x_hbm.at[pl.ds(g0, chunk), ift, wi, pl.ds(il8, 8)],
                    blocks.at[wi],
                    sem,
                )
                for wi in range(wsub)
            ]
            for d in descs:
                d.start()
            for d in descs:
                d.wait()
            lane16 = lax.iota(jnp.int32, 16)
            for wi in range(wsub):
                coli = jnp.full((16,), wi, jnp.int32)
                lanei = jnp.full((16,), lane_in_block, jnp.int32)
                for j in range(chunk // 16):
                    rows = j * 16 + lane16
                    vals = plsc.load_gather(blocks.at[wi], [rows, lanei])
                    plsc.store_scatter(colt, [rows, coli], vals)
            pltpu.sync_copy(colt, out_hbm.at[pl.ds(g0, chunk)])

    return sc_gather


def kernel(x):
    frames, chn, hgt, wdt = x.shape
    choices = _select_indices(frames)
    ch = choices[0]
    flanes = 128
    wsub = 8
    ftiles = frames // flanes
    wtiles = wdt // wsub
    n_groups = chn * hgt * wtiles
    xv = (
        x.reshape(ftiles, flanes, chn, hgt, wtiles, wsub)
        .transpose(2, 3, 4, 0, 5, 1)
        .reshape(n_groups, ftiles, wsub, flanes)
    )
    sc_gather = _make_sc_gather(
        n_groups, ftiles, wsub, flanes, ch // flanes, ch % flanes
    )
    out2 = sc_gather(xv)
    return out2.reshape(1, chn, hgt, wdt)


# speedup vs baseline: 11.6983x; 1.0472x over previous
import functools

import numpy as np
import jax
import jax.numpy as jnp
from jax import lax
from jax.experimental import pallas as pl
from jax.experimental.pallas import tpu as pltpu
from jax.experimental.pallas import tpu_sc as plsc


def _norm_pdf_np(z):
    return np.exp(-0.5 * z * z) / np.sqrt(2.0 * np.pi)


def _select_indices(frame_count: int) -> list:
    rng = np.random.RandomState(0)
    num_segments = 1
    idxs = np.linspace(0, frame_count - 1, frame_count, dtype=int)
    if frame_count <= num_segments * 2:
        idxs = np.repeat(idxs, int(frame_count * num_segments / len(idxs)))
        frame_count *= num_segments
    seg_sizes = _norm_pdf_np(np.linspace(-1, 1, num_segments))
    seg_sizes = 1 - seg_sizes if frame_count > num_segments else seg_sizes
    seg_sizes = seg_sizes / seg_sizes.sum() * frame_count
    seg_sizes = seg_sizes.astype(int)
    choices = []
    last_idx = 0
    for i, seg_size in enumerate(seg_sizes):
        next_idx = last_idx + seg_size if i < len(seg_sizes) - 1 else None
        choices.append(int(rng.choice(idxs[last_idx:next_idx], 1)[0]))
        last_idx = next_idx
    return choices


@functools.lru_cache(maxsize=None)
def _make_sc_gather(n_groups: int, ftiles: int, wsub: int, flanes: int,
                    ift: int, ilane: int):
    info = plsc.get_sparse_core_info()
    nw = info.num_cores * info.num_subcores
    while nw > 1 and (n_groups % nw != 0 or (n_groups // nw) % 8 != 0):
        nw -= 1
    chunk = n_groups // nw
    mesh = plsc.VectorSubcoreMesh(core_axis_name="c", subcore_axis_name="s")

    il8 = (ilane // 8) * 8
    lane_in_block = ilane - il8
    assert chunk % 16 == 0

    @functools.partial(
        pl.kernel,
        mesh=mesh,
        out_type=jax.ShapeDtypeStruct((n_groups, wsub), jnp.float32),
        scratch_types=[
            pltpu.VMEM((wsub, chunk, 8), jnp.float32),
            pltpu.VMEM((chunk, wsub), jnp.float32),
            pltpu.SemaphoreType.DMA,
        ],
        compiler_params=pltpu.CompilerParams(
            use_tc_tiling_on_sc=False, needs_layout_passes=False
        ),
    )
    def sc_gather(x_hbm, out_hbm, blocks, colt, sem):
        wid = lax.axis_index("s") * info.num_cores + lax.axis_index("c")
        g0 = wid * chunk

        @pl.when(wid < nw)
        def _():
            descs = [
                pltpu.make_async_copy(
                    x_hbm.at[pl.ds(g0, chunk), ift, wi, pl.ds(il8, 8)],
                    blocks.at[wi],
                    sem,
                )
                for wi in range(wsub)
            ]
            for d in descs:
                d.start()
            lane16 = lax.iota(jnp.int32, 16)
            lanei = jnp.full((16,), lane_in_block, jnp.int32)
            for wi in range(wsub):
                descs[wi].wait()
                coli = jnp.full((16,), wi, jnp.int32)
                for j in range(chunk // 16):
                    rows = j * 16 + lane16
                    vals = plsc.load_gather(blocks.at[wi], [rows, lanei])
                    plsc.store_scatter(colt, [rows, coli], vals)
            pltpu.sync_copy(colt, out_hbm.at[pl.ds(g0, chunk)])

    return sc_gather


def kernel(x):
    frames, chn, hgt, wdt = x.shape
    choices = _select_indices(frames)
    ch = choices[0]
    flanes = 128
    wsub = 8
    ftiles = frames // flanes
    wtiles = wdt // wsub
    n_groups = chn * hgt * wtiles
    xv = (
        x.reshape(ftiles, flanes, chn, hgt, wtiles, wsub)
        .transpose(2, 3, 4, 0, 5, 1)
        .reshape(n_groups, ftiles, wsub, flanes)
    )
    sc_gather = _make_sc_gather(
        n_groups, ftiles, wsub, flanes, ch // flanes, ch % flanes
    )
    out2 = sc_gather(xv)
    return out2.reshape(1, chn, hgt, wdt)
